# Initial kernel scaffold; baseline (speedup 1.0000x reference)
import jax
import jax.numpy as jnp
from jax.experimental import pallas as pl


def kernel(hid, gate_W, u, g, d):
    raise NotImplementedError("write your pallas kernel here")


# speedup vs baseline: 1.4243x; 1.4243x over previous
import functools

import jax
import jax.numpy as jnp
from jax.experimental import pallas as pl

NUM_EXPERTS = 16
EXPERT_W = 512
TOP_K = 2
HID_DIM = 2048
INTER_DIM = NUM_EXPERTS * EXPERT_W
N_TOKENS = 4096

ROW_BLK = 512


def _router_body(x_ref, gw_ref, rw_ref):
    logits = jax.lax.dot_general(
        x_ref[...], gw_ref[...], (((1,), (1,)), ((), ())),
        preferred_element_type=jnp.float32)
    p = jax.nn.softmax(logits, axis=1)
    ii = jax.lax.broadcasted_iota(jnp.int32, p.shape, 1)
    w1 = jnp.max(p, axis=1, keepdims=True)
    i1 = jnp.min(jnp.where(p == w1, ii, NUM_EXPERTS), axis=1, keepdims=True)
    m1 = ii == i1
    p2 = jnp.where(m1, -1.0, p)
    w2 = jnp.max(p2, axis=1, keepdims=True)
    i2 = jnp.min(jnp.where(p2 == w2, ii, NUM_EXPERTS), axis=1, keepdims=True)
    m2 = ii == i2
    rw_ref[...] = jnp.where(m1 | m2, p, 0.0)


def _router(x, gate_W):
    grid = (N_TOKENS // ROW_BLK,)
    return pl.pallas_call(
        _router_body,
        grid=grid,
        in_specs=[
            pl.BlockSpec((ROW_BLK, HID_DIM), lambda i: (i, 0)),
            pl.BlockSpec((NUM_EXPERTS, HID_DIM), lambda i: (0, 0)),
        ],
        out_specs=pl.BlockSpec((ROW_BLK, NUM_EXPERTS), lambda i: (i, 0)),
        out_shape=jax.ShapeDtypeStruct((N_TOKENS, NUM_EXPERTS), jnp.float32),
    )(x, gate_W)


def _moe_body(x_ref, u_ref, g_ref, d_ref, rw_ref, out_ref):
    e = pl.program_id(1)
    x = x_ref[...]
    up = jax.lax.dot_general(x, u_ref[...], (((1,), (1,)), ((), ())),
                             preferred_element_type=jnp.float32)
    gt = jax.lax.dot_general(x, g_ref[...], (((1,), (1,)), ((), ())),
                             preferred_element_type=jnp.float32)
    ir = up * (gt * jax.nn.sigmoid(gt))
    ii = jax.lax.broadcasted_iota(jnp.int32, rw_ref.shape, 1)
    w_col = jnp.sum(jnp.where(ii == e, rw_ref[...], 0.0), axis=1, keepdims=True)
    ir = ir * w_col
    contrib = jax.lax.dot_general(ir, d_ref[...], (((1,), (1,)), ((), ())),
                                  preferred_element_type=jnp.float32)

    @pl.when(e == 0)
    def _():
        out_ref[...] = contrib

    @pl.when(e > 0)
    def _():
        out_ref[...] += contrib


def _moe_dense(x, u, g, d, rw):
    grid = (N_TOKENS // ROW_BLK, NUM_EXPERTS)
    return pl.pallas_call(
        _moe_body,
        grid=grid,
        in_specs=[
            pl.BlockSpec((ROW_BLK, HID_DIM), lambda i, e: (i, 0)),
            pl.BlockSpec((EXPERT_W, HID_DIM), lambda i, e: (e, 0)),
            pl.BlockSpec((EXPERT_W, HID_DIM), lambda i, e: (e, 0)),
            pl.BlockSpec((HID_DIM, EXPERT_W), lambda i, e: (0, e)),
            pl.BlockSpec((ROW_BLK, NUM_EXPERTS), lambda i, e: (i, 0)),
        ],
        out_specs=pl.BlockSpec((ROW_BLK, HID_DIM), lambda i, e: (i, 0)),
        out_shape=jax.ShapeDtypeStruct((N_TOKENS, HID_DIM), jnp.float32),
    )(x, u, g, d, rw)


def kernel(hid, gate_W, u, g, d):
    x = hid.reshape(-1, HID_DIM)
    rw = _router(x, gate_W)
    return _moe_dense(x, u, g, d, rw)


# speedup vs baseline: 2.1564x; 1.5140x over previous
import functools

import jax
import jax.numpy as jnp
from jax import lax
from jax.experimental import pallas as pl
from jax.experimental.pallas import tpu as pltpu, tpu_sc as plsc

NUM_EXPERTS = 16
EXPERT_W = 512
HID_DIM = 2048
N_TOKENS = 4096
N_ASSIGN = 2 * N_TOKENS

NW = 32
TPW = N_TOKENS // NW
L = 16

ROW_BLK = 512
GTILE = 256
NTILES = N_ASSIGN // GTILE
NSTEPS = NTILES + NUM_EXPERTS

_MESH = dict(core_axis_name="c", subcore_axis_name="s")


def _wid():
    return lax.axis_index("s") * 2 + lax.axis_index("c")


def _router_body(x_ref, gw_ref, rw_ref):
    logits = lax.dot_general(x_ref[...], gw_ref[...], (((1,), (1,)), ((), ())),
                             preferred_element_type=jnp.float32)
    p = jax.nn.softmax(logits, axis=1)
    ii = lax.broadcasted_iota(jnp.int32, p.shape, 1)
    w1 = jnp.max(p, axis=1, keepdims=True)
    i1 = jnp.min(jnp.where(p == w1, ii, NUM_EXPERTS), axis=1, keepdims=True)
    m1 = ii == i1
    p2 = jnp.where(m1, -1.0, p)
    w2 = jnp.max(p2, axis=1, keepdims=True)
    i2 = jnp.min(jnp.where(p2 == w2, ii, NUM_EXPERTS), axis=1, keepdims=True)
    rw_ref[...] = jnp.where(m1 | (ii == i2), p, 0.0)


def _router(x, gate_W):
    return pl.pallas_call(
        _router_body,
        grid=(N_TOKENS // ROW_BLK,),
        in_specs=[
            pl.BlockSpec((ROW_BLK, HID_DIM), lambda i: (i, 0)),
            pl.BlockSpec((NUM_EXPERTS, HID_DIM), lambda i: (0, 0)),
        ],
        out_specs=pl.BlockSpec((ROW_BLK, NUM_EXPERTS), lambda i: (i, 0)),
        out_shape=jax.ShapeDtypeStruct((N_TOKENS, NUM_EXPERTS), jnp.float32),
    )(x, gate_W)


def _extract_body(rw_hbm, eids_hbm, topw_hbm, hist_hbm,
                  rw_v, e0_v, e1_v, w0_v, w1_v, hist_v):
    wid = _wid()
    pltpu.sync_copy(rw_hbm.at[pl.ds(wid * TPW, TPW)], rw_v)
    lane = lax.iota(jnp.int32, L)
    hist = jnp.zeros((L,), jnp.int32)
    for c in range(TPW // L):
        rows = jnp.full((L,), c * L, jnp.int32) + lane
        i1 = jnp.zeros((L,), jnp.int32)
        i2 = jnp.zeros((L,), jnp.int32)
        w1 = jnp.full((L,), -1.0, jnp.float32)
        w2 = jnp.full((L,), -1.0, jnp.float32)
        for e in range(NUM_EXPERTS):
            col = plsc.load_gather(rw_v, [rows, jnp.full((L,), e, jnp.int32)])
            up1 = col > w1
            up2 = jnp.logical_and(jnp.logical_not(up1), col > w2)
            i2 = jnp.where(up1, i1, jnp.where(up2, e, i2))
            w2 = jnp.where(up1, w1, jnp.where(up2, col, w2))
            i1 = jnp.where(up1, e, i1)
            w1 = jnp.where(up1, col, w1)
        e0_v[pl.ds(c * L, L)] = i1
        e1_v[pl.ds(c * L, L)] = i2
        w0_v[pl.ds(c * L, L)] = w1
        w1_v[pl.ds(c * L, L)] = w2
        for e in range(NUM_EXPERTS):
            cnt = (plsc.all_reduce_population_count(i1 == e)
                   + plsc.all_reduce_population_count(i2 == e))
            hist = hist + jnp.where(lane == e, cnt, 0)
    hist_v[...] = hist
    pltpu.sync_copy(e0_v, eids_hbm.at[0, wid])
    pltpu.sync_copy(e1_v, eids_hbm.at[1, wid])
    pltpu.sync_copy(w0_v, topw_hbm.at[0, wid])
    pltpu.sync_copy(w1_v, topw_hbm.at[1, wid])
    pltpu.sync_copy(hist_v, hist_hbm.at[wid])


def _sc_extract(rw):
    f = pl.kernel(
        _extract_body,
        out_type=[
            jax.ShapeDtypeStruct((2, NW, TPW), jnp.int32),
            jax.ShapeDtypeStruct((2, NW, TPW), jnp.float32),
            jax.ShapeDtypeStruct((NW, NUM_EXPERTS), jnp.int32),
        ],
        mesh=plsc.VectorSubcoreMesh(**_MESH),
        compiler_params=pltpu.CompilerParams(needs_layout_passes=False),
        scratch_types=[
            pltpu.VMEM((TPW, NUM_EXPERTS), jnp.float32),
            pltpu.VMEM((TPW,), jnp.int32),
            pltpu.VMEM((TPW,), jnp.int32),
            pltpu.VMEM((TPW,), jnp.float32),
            pltpu.VMEM((TPW,), jnp.float32),
            pltpu.VMEM((NUM_EXPERTS,), jnp.int32),
        ],
    )
    return f(rw)


def _place_body(eids_hbm, hist_hbm, topw_hbm, pos0_hbm, pos1_hbm, stok_hbm,
                sortw_hbm, hist_v, e0_v, e1_v, p0_v, p1_v, t0_v, t1_v,
                w0_v, w1_v, sem):
    wid = _wid()
    pltpu.sync_copy(hist_hbm, hist_v)
    pltpu.sync_copy(eids_hbm.at[0, wid], e0_v)
    pltpu.sync_copy(eids_hbm.at[1, wid], e1_v)
    pltpu.sync_copy(topw_hbm.at[0, wid], w0_v)
    pltpu.sync_copy(topw_hbm.at[1, wid], w1_v)
    lane = lax.iota(jnp.int32, L)
    tot = jnp.zeros((L,), jnp.int32)
    pre = jnp.zeros((L,), jnp.int32)
    for w in range(NW):
        row = hist_v[w, :]
        tot = tot + row
        pre = pre + row * (jnp.full((L,), w, jnp.int32) < wid).astype(jnp.int32)
    base = (plsc.cumsum(tot) - tot) + pre
    for k in range(2):
        e_v = e0_v if k == 0 else e1_v
        p_v = p0_v if k == 0 else p1_v
        t_v = t0_v if k == 0 else t1_v
        for c in range(TPW // L):
            chunk = e_v[pl.ds(c * L, L)]
            pos_c = jnp.zeros((L,), jnp.int32)
            for e in range(NUM_EXPERTS):
                m = chunk == e
                pref = plsc.cumsum(m.astype(jnp.int32))
                b_e = jnp.sum(jnp.where(lane == e, base, 0))
                pos_c = jnp.where(m, b_e + pref - 1, pos_c)
                cnt = plsc.all_reduce_population_count(m)
                base = base + jnp.where(lane == e, cnt, 0)
            p_v[pl.ds(c * L, L)] = pos_c
            t_v[pl.ds(c * L, L)] = wid * TPW + c * L + lane
    pltpu.sync_copy(p0_v, pos0_hbm.at[wid])
    pltpu.sync_copy(p1_v, pos1_hbm.at[wid])
    pltpu.async_copy(t0_v, stok_hbm.at[p0_v], sem).wait()
    pltpu.async_copy(t1_v, stok_hbm.at[p1_v], sem).wait()
    pltpu.async_copy(w0_v, sortw_hbm.at[p0_v], sem).wait()
    pltpu.async_copy(w1_v, sortw_hbm.at[p1_v], sem).wait()


def _sc_place(eids, hist, topw):
    f = pl.kernel(
        _place_body,
        out_type=[
            jax.ShapeDtypeStruct((NW, TPW), jnp.int32),
            jax.ShapeDtypeStruct((NW, TPW), jnp.int32),
            jax.ShapeDtypeStruct((N_ASSIGN,), jnp.int32),
            jax.ShapeDtypeStruct((N_ASSIGN,), jnp.float32),
        ],
        mesh=plsc.VectorSubcoreMesh(**_MESH),
        compiler_params=pltpu.CompilerParams(needs_layout_passes=False),
        scratch_types=[
            pltpu.VMEM((NW, NUM_EXPERTS), jnp.int32),
            pltpu.VMEM((TPW,), jnp.int32),
            pltpu.VMEM((TPW,), jnp.int32),
            pltpu.VMEM((TPW,), jnp.int32),
            pltpu.VMEM((TPW,), jnp.int32),
            pltpu.VMEM((TPW,), jnp.int32),
            pltpu.VMEM((TPW,), jnp.int32),
            pltpu.VMEM((TPW,), jnp.float32),
            pltpu.VMEM((TPW,), jnp.float32),
            pltpu.SemaphoreType.DMA,
        ],
    )
    return f(eids, hist, topw)


_GC = 16


def _gather_body(x_hbm, stok_hbm, xs_hbm, idx_v, buf0, buf1, gs0, gs1, ws0, ws1):
    wid = _wid()
    rpw = N_ASSIGN // NW
    base = wid * rpw
    nchunk = rpw // _GC
    bufs = (buf0, buf1)
    gsems = (gs0, gs1)
    wsems = (ws0, ws1)
    pltpu.sync_copy(stok_hbm.at[pl.ds(base, rpw)], idx_v)

    def start_gather(c):
        ivec = idx_v[pl.ds(c * _GC, _GC)]
        return pltpu.async_copy(x_hbm.at[ivec], bufs[c % 2], gsems[c % 2])

    gd = start_gather(0)
    wd_prev = None
    for c in range(nchunk):
        if c + 1 < nchunk:
            if wd_prev is not None:
                wd_prev.wait()
            gd_next = start_gather(c + 1)
        else:
            gd_next = None
            if wd_prev is not None:
                wd_prev.wait()
        gd.wait()
        wd_prev = pltpu.async_copy(
            bufs[c % 2], xs_hbm.at[pl.ds(base + c * _GC, _GC)], wsems[c % 2])
        gd = gd_next
    wd_prev.wait()


def _sc_gather(x, stok):
    f = pl.kernel(
        _gather_body,
        out_type=jax.ShapeDtypeStruct((N_ASSIGN, HID_DIM), jnp.float32),
        mesh=plsc.VectorSubcoreMesh(**_MESH),
        compiler_params=pltpu.CompilerParams(needs_layout_passes=False),
        scratch_types=[
            pltpu.VMEM((N_ASSIGN // NW,), jnp.int32),
            pltpu.VMEM((_GC, HID_DIM), jnp.float32),
            pltpu.VMEM((_GC, HID_DIM), jnp.float32),
            pltpu.SemaphoreType.DMA,
            pltpu.SemaphoreType.DMA,
            pltpu.SemaphoreType.DMA,
            pltpu.SemaphoreType.DMA,
        ],
    )
    return f(x, stok)


def _gmm_body(m_ref, xs_ref, u_ref, g_ref, d_ref, sw_ref, o_ref):
    s = pl.program_id(0)
    lo = m_ref[s, 2]
    hi = m_ref[s, 3]
    first = m_ref[s, 4]

    @pl.when(hi > lo)
    def _():
        x = xs_ref[...]
        up = lax.dot_general(x, u_ref[...], (((1,), (1,)), ((), ())),
                             preferred_element_type=jnp.float32)
        gt = lax.dot_general(x, g_ref[...], (((1,), (1,)), ((), ())),
                             preferred_element_type=jnp.float32)
        ir = up * (gt * jax.nn.sigmoid(gt))
        ir = ir * sw_ref[0]
        rows = lax.broadcasted_iota(jnp.int32, ir.shape, 0)
        ir = jnp.where((rows >= lo) & (rows < hi), ir, 0.0)
        contrib = lax.dot_general(ir, d_ref[...], (((1,), (1,)), ((), ())),
                                  preferred_element_type=jnp.float32)

        @pl.when(first == 1)
        def _():
            o_ref[...] = contrib

        @pl.when(first == 0)
        def _():
            o_ref[...] += contrib


def _tc_grouped(xs, u, g, d, sortw, meta):
    grid_spec = pltpu.PrefetchScalarGridSpec(
        num_scalar_prefetch=1,
        grid=(NSTEPS,),
        in_specs=[
            pl.BlockSpec((GTILE, HID_DIM), lambda s, m: (m[s, 0], 0)),
            pl.BlockSpec((EXPERT_W, HID_DIM), lambda s, m: (m[s, 1], 0)),
            pl.BlockSpec((EXPERT_W, HID_DIM), lambda s, m: (m[s, 1], 0)),
            pl.BlockSpec((HID_DIM, EXPERT_W), lambda s, m: (0, m[s, 1])),
            pl.BlockSpec((1, GTILE, 1), lambda s, m: (m[s, 0], 0, 0)),
        ],
        out_specs=pl.BlockSpec((GTILE, HID_DIM), lambda s, m: (m[s, 0], 0)),
    )
    return pl.pallas_call(
        _gmm_body,
        grid_spec=grid_spec,
        out_shape=jax.ShapeDtypeStruct((N_ASSIGN, HID_DIM), jnp.float32),
        compiler_params=pltpu.CompilerParams(
            dimension_semantics=("arbitrary",),
            vmem_limit_bytes=100 * 1024 * 1024,
        ),
    )(meta, xs, u, g, d, sortw.reshape(NTILES, GTILE, 1))


def _make_meta(hist):
    tot = jnp.sum(hist, axis=0)
    b = jnp.concatenate([jnp.zeros((1,), jnp.int32),
                         jnp.cumsum(tot, dtype=jnp.int32)])
    tiles = jnp.arange(NTILES, dtype=jnp.int32)[:, None]
    valid = ((b[1:][None, :] > tiles * GTILE)
             & (b[:-1][None, :] < (tiles + 1) * GTILE)
             & (tot[None, :] > 0))
    flat = valid.reshape(-1)
    order = jnp.argsort(jnp.logical_not(flat), stable=True)
    steps = order[:NSTEPS]
    vstep = flat[steps]
    tile = jnp.where(vstep, steps // NUM_EXPERTS, NTILES - 1).astype(jnp.int32)
    expert = jnp.where(vstep, steps % NUM_EXPERTS, 0).astype(jnp.int32)
    lo = jnp.clip(b[expert] - tile * GTILE, 0, GTILE)
    hi = jnp.clip(b[expert + 1] - tile * GTILE, 0, GTILE)
    lo = jnp.where(vstep, lo, 0).astype(jnp.int32)
    hi = jnp.where(vstep, hi, 0).astype(jnp.int32)
    prev_tile = jnp.concatenate([jnp.full((1,), -1, jnp.int32), tile[:-1]])
    first = (vstep & (tile != prev_tile)).astype(jnp.int32)
    return jnp.stack([tile, expert, lo, hi, first], axis=1)


_CC = 8


def _combine_body(os_hbm, pos0_hbm, pos1_hbm, a_hbm, b_hbm,
                  p0_v, p1_v, bufa0, bufb0, bufa1, bufb1,
                  ga0, gb0, ga1, gb1, wa0, wb0, wa1, wb1):
    wid = _wid()
    pltpu.sync_copy(pos0_hbm.at[wid], p0_v)
    pltpu.sync_copy(pos1_hbm.at[wid], p1_v)
    bufa = (bufa0, bufa1)
    bufb = (bufb0, bufb1)
    gsa = (ga0, ga1)
    gsb = (gb0, gb1)
    wsa = (wa0, wa1)
    wsb = (wb0, wb1)
    nchunk = TPW // _CC
    wda = [None] * nchunk
    wdb = [None] * nchunk
    for c in range(nchunk):
        p = c % 2
        if c >= 2:
            wda[c - 2].wait()
            wdb[c - 2].wait()
        i0 = p0_v.at[pl.ds(c * _CC, _CC)]
        i1 = p1_v.at[pl.ds(c * _CC, _CC)]
        da = pltpu.async_copy(os_hbm.at[i0], bufa[p], gsa[p])
        db = pltpu.async_copy(os_hbm.at[i1], bufb[p], gsb[p])
        da.wait()
        db.wait()
        rows = pl.ds(wid * TPW + c * _CC, _CC)
        wda[c] = pltpu.async_copy(bufa[p], a_hbm.at[rows], wsa[p])
        wdb[c] = pltpu.async_copy(bufb[p], b_hbm.at[rows], wsb[p])
    for c in (nchunk - 2, nchunk - 1):
        wda[c].wait()
        wdb[c].wait()


def _sc_combine(out_s, pos0, pos1):
    f = pl.kernel(
        _combine_body,
        out_type=[
            jax.ShapeDtypeStruct((N_TOKENS, HID_DIM), jnp.float32),
            jax.ShapeDtypeStruct((N_TOKENS, HID_DIM), jnp.float32),
        ],
        mesh=plsc.VectorSubcoreMesh(**_MESH),
        compiler_params=pltpu.CompilerParams(needs_layout_passes=False),
        scratch_types=[
            pltpu.VMEM((TPW,), jnp.int32),
            pltpu.VMEM((TPW,), jnp.int32),
            pltpu.VMEM((_CC, HID_DIM), jnp.float32),
            pltpu.VMEM((_CC, HID_DIM), jnp.float32),
            pltpu.VMEM((_CC, HID_DIM), jnp.float32),
            pltpu.VMEM((_CC, HID_DIM), jnp.float32),
        ] + [pltpu.SemaphoreType.DMA] * 8,
    )
    return f(out_s, pos0, pos1)


def _add_body(a_ref, b_ref, o_ref):
    o_ref[...] = a_ref[...] + b_ref[...]


def _tc_add(a, b):
    blk = 512
    return pl.pallas_call(
        _add_body,
        grid=(N_TOKENS // blk,),
        in_specs=[
            pl.BlockSpec((blk, HID_DIM), lambda i: (i, 0)),
            pl.BlockSpec((blk, HID_DIM), lambda i: (i, 0)),
        ],
        out_specs=pl.BlockSpec((blk, HID_DIM), lambda i: (i, 0)),
        out_shape=jax.ShapeDtypeStruct((N_TOKENS, HID_DIM), jnp.float32),
    )(a, b)


def kernel(hid, gate_W, u, g, d):
    x = hid.reshape(-1, HID_DIM)
    rw = _router(x, gate_W)
    eids, topw, hist = _sc_extract(rw)
    pos0, pos1, stok, sortw = _sc_place(eids, hist, topw)
    xs = _sc_gather(x, stok)
    meta = _make_meta(hist)
    out_s = _tc_grouped(xs, u, g, d, sortw, meta)
    a, b = _sc_combine(out_s, pos0, pos1)
    return _tc_add(a, b)
